# Initial kernel scaffold; baseline (speedup 1.0000x reference)
import jax
import jax.numpy as jnp
from jax.experimental import pallas as pl


def kernel(samples, theta, kernel, observables):
    raise NotImplementedError("write your pallas kernel here")


# speedup vs baseline: 2.1337x; 2.1337x over previous
import functools

import jax
import jax.numpy as jnp
from jax import lax
from jax.experimental import pallas as pl
from jax.experimental.pallas import tpu as pltpu
from jax.experimental.pallas import tpu_sc as plsc

_N_BITS = 20
_N_OPS = 64
_NUM_SAMPLES = 1048576
_NUM_BINS = 1 << _N_BITS


_BLKL = 32768
_GRID = _NUM_SAMPLES // _BLKL


def _tc_body(samples_ref, theta_ref, w_ref, obs_ref, codes_ref, exp_ref,
             m_ref, s_ref, v_ref):
    step = pl.program_id(0)

    xi = samples_ref[...]
    x = xi.astype(jnp.float32)

    theta = theta_ref[...]
    w = w_ref[...]
    ii = lax.broadcasted_iota(jnp.int32, (_N_BITS, _N_BITS), 0)
    jj = lax.broadcasted_iota(jnp.int32, (_N_BITS, _N_BITS), 1)
    wp = w + jnp.where(ii == jj, jnp.broadcast_to(theta, (_N_BITS, _N_BITS)), 0.0)

    wx = lax.dot_general(wp, x, (((1,), (0,)), ((), ())),
                         preferred_element_type=jnp.float32)
    logit = jnp.sum(wx * x, axis=0, keepdims=True)

    powers = jnp.left_shift(
        jnp.int32(1), lax.broadcasted_iota(jnp.int32, (_N_BITS, 1), 0))
    codes = jnp.sum(xi * powers, axis=0)
    codes_ref[...] = codes

    @pl.when(step == 0)
    def _init():
        m_ref[0] = -jnp.inf
        s_ref[0] = 0.0
        v_ref[...] = jnp.zeros_like(v_ref)

    m_old = m_ref[0]
    bm = jnp.max(logit)
    m_new = jnp.maximum(m_old, bm)
    scale = jnp.exp(m_old - m_new)
    e = jnp.exp(logit - m_new)
    s_new = s_ref[0] * scale + jnp.sum(e)
    ev = lax.dot_general(x, e, (((1,), (1,)), ((), ())),
                         preferred_element_type=jnp.float32)
    v_new = v_ref[0:_N_BITS, 0:1] * scale + ev
    m_ref[0] = m_new
    s_ref[0] = s_new
    v_ref[0:_N_BITS, 0:1] = v_new

    zbar = 1.0 - 2.0 * v_new / s_new
    obs = obs_ref[...]
    exp_ref[...] = lax.dot_general(obs, zbar, (((1,), (0,)), ((), ())),
                                   preferred_element_type=jnp.float32)


def _tc_stage(samples_t, theta, w, obs):
    codes, exps = pl.pallas_call(
        _tc_body,
        grid=(_GRID,),
        in_specs=[
            pl.BlockSpec((_N_BITS, _BLKL), lambda i: (0, i)),
            pl.BlockSpec((1, _N_BITS), lambda i: (0, 0)),
            pl.BlockSpec((_N_BITS, _N_BITS), lambda i: (0, 0)),
            pl.BlockSpec((_N_OPS, _N_BITS), lambda i: (0, 0)),
        ],
        out_specs=[
            pl.BlockSpec((_BLKL,), lambda i: (i,)),
            pl.BlockSpec((_N_OPS, 1), lambda i: (0, 0)),
        ],
        out_shape=[
            jax.ShapeDtypeStruct((_NUM_SAMPLES,), jnp.int32),
            jax.ShapeDtypeStruct((_N_OPS, 1), jnp.float32),
        ],
        scratch_shapes=[
            pltpu.SMEM((1,), jnp.float32),
            pltpu.SMEM((1,), jnp.float32),
            pltpu.VMEM((_N_BITS, 1), jnp.float32),
        ],
    )(samples_t, theta.reshape(1, _N_BITS), w, obs)
    return codes, exps.reshape(_N_OPS)


_N_TILES = 16
_PER_TILE = _NUM_SAMPLES // _N_TILES
_IDX_ROWS = _PER_TILE // 128
_CHUNK_ROWS = 64
_N_CHUNKS = _IDX_ROWS // _CHUNK_ROWS
_BINS_PER_TILE = _NUM_BINS // _N_TILES
_ZCHUNK = 2048


def _sc_hist_body(codes_hbm, out_hbm, table, idx_v, ones_v, zeros_v):
    cid = lax.axis_index("c")
    sid = lax.axis_index("s")

    @pl.when(cid == 0)
    def _active():
        def fillz(i, _):
            zeros_v[pl.ds(i * 16, 16)] = jnp.zeros((16,), jnp.int32)
            return 0
        lax.fori_loop(0, _ZCHUNK // 16, fillz, 0)

        def fill1(i, _):
            ones_v[pl.ds(i * 16, 16)] = jnp.ones((16,), jnp.int32)
            return 0
        lax.fori_loop(0, 128 // 16, fill1, 0)

        def zslice(j, _):
            pltpu.sync_copy(
                zeros_v,
                table.at[pl.ds(sid * _BINS_PER_TILE + j * _ZCHUNK, _ZCHUNK)])
            return 0
        lax.fori_loop(0, _BINS_PER_TILE // _ZCHUNK, zslice, 0)
        plsc.subcore_barrier()

        def chunk(c, _):
            pltpu.sync_copy(
                codes_hbm.at[pl.ds(sid * _IDX_ROWS + c * _CHUNK_ROWS,
                                   _CHUNK_ROWS)],
                idx_v)

            def scat(j, _):
                pltpu.sync_copy(ones_v, table.at[idx_v.at[j]], add=True)
                return 0
            lax.fori_loop(0, _CHUNK_ROWS, scat, 0)
            return 0
        lax.fori_loop(0, _N_CHUNKS, chunk, 0)
        plsc.subcore_barrier()

        pltpu.sync_copy(
            table.at[pl.ds(sid * _BINS_PER_TILE, _BINS_PER_TILE)],
            out_hbm.at[pl.ds(sid * _BINS_PER_TILE, _BINS_PER_TILE)])


def _sc_hist(codes):
    k = pl.kernel(
        _sc_hist_body,
        out_type=jax.ShapeDtypeStruct((_NUM_BINS,), jnp.int32),
        mesh=plsc.VectorSubcoreMesh(core_axis_name="c", subcore_axis_name="s"),
        scratch_types=[
            pltpu.VMEM_SHARED((_NUM_BINS,), jnp.int32),
            pltpu.VMEM((_CHUNK_ROWS, 128), jnp.int32),
            pltpu.VMEM((128,), jnp.int32),
            pltpu.VMEM((_ZCHUNK,), jnp.int32),
        ],
    )
    return k(codes.reshape(_NUM_SAMPLES // 128, 128))


def kernel(samples, theta, kernel, observables):
    codes, expectations = _tc_stage(samples.T, theta, kernel, observables)
    counts = _sc_hist(codes)
    return counts, expectations
